# baseline (speedup 1.0000x reference)
import functools

import jax
import jax.numpy as jnp
from jax import lax
from jax.experimental import pallas as pl
from jax.experimental.pallas import tpu as pltpu
from jax.experimental.pallas import tpu_sc as plsc

B = 16384
D = 128
NC = 2
NS = 16
NW = NC * NS
BPW = B // NW
CH = 64
NCH = BPW // CH
LANES = 16


def _prod_chunk(u_ref, i_ref, p_ref):
    def row(r, _):
        for k in range(D // LANES):
            sl = pl.ds(LANES * k, LANES)
            p_ref[r, sl] = u_ref[r, sl] * i_ref[r, sl]
        return 0

    lax.fori_loop(0, CH, row, 0, unroll=2)


def _make_gather_body(nch):
    bpw = nch * CH
    return functools.partial(_gather_body_merged, nch, bpw)


def _gather_body_merged(NCH, BPW, uid_ref, iid_ref, ug_t, ig_t, um_t, im_t,
                        pr_o, um_o, im_o,
                        uidx_v, iidx_v, bu, bi, bp, bm, bn, gsem, ssem):
    wid = lax.axis_index("s") * NC + lax.axis_index("c")
    base = wid * BPW
    pltpu.sync_copy(uid_ref.at[wid], uidx_v)
    pltpu.sync_copy(iid_ref.at[wid], iidx_v)

    njobs = 2 * NCH
    gath = [None] * njobs
    scat = [None] * njobs

    def issue_gather(j):
        c, s = j // 2, (j // 2) % 2
        if j % 2 == 0:
            return [pltpu.async_copy(ug_t.at[uidx_v.at[c]], bu.at[s], gsem),
                    pltpu.async_copy(ig_t.at[iidx_v.at[c]], bi.at[s], gsem)]
        return [pltpu.async_copy(um_t.at[uidx_v.at[c]], bm.at[s], gsem),
                pltpu.async_copy(im_t.at[iidx_v.at[c]], bn.at[s], gsem)]

    def issue_scatter(j):
        c, s = j // 2, (j // 2) % 2
        dst = pl.ds(base + c * CH, CH)
        if j % 2 == 0:
            return [pltpu.async_copy(bp.at[s], pr_o.at[dst], ssem)]
        return [pltpu.async_copy(bm.at[s], um_o.at[dst], ssem),
                pltpu.async_copy(bn.at[s], im_o.at[dst], ssem)]

    gath[0] = issue_gather(0)
    if njobs > 1:
        gath[1] = issue_gather(1)
    for j in range(njobs):
        nj = j + 2
        if nj < njobs:
            if nj - 4 >= 0 and scat[nj - 4] is not None:
                for h in scat[nj - 4]:
                    h.wait()
                scat[nj - 4] = None
            gath[nj] = issue_gather(nj)
        for h in gath[j]:
            h.wait()
        if j % 2 == 0:
            s = (j // 2) % 2
            if j - 4 >= 0 and scat[j - 4] is not None:
                for h in scat[j - 4]:
                    h.wait()
                scat[j - 4] = None
            _prod_chunk(bu.at[s], bi.at[s], bp.at[s])
        scat[j] = issue_scatter(j)
    for hs in scat:
        if hs is not None:
            for h in hs:
                h.wait()


def _sc_gather(user_ids, item_ids, ue_gmf, ie_gmf, ue_mlp, ie_mlp):
    nb = user_ids.shape[0]
    nch = nb // (NW * CH)
    mesh = plsc.VectorSubcoreMesh(core_axis_name="c", subcore_axis_name="s",
                                  num_cores=NC, num_subcores=NS)
    f = pl.kernel(
        _make_gather_body(nch),
        out_type=[jax.ShapeDtypeStruct((nb, D), jnp.float32)] * 3,
        mesh=mesh,
        scratch_types=[
            pltpu.VMEM((nch, CH), jnp.int32),
            pltpu.VMEM((nch, CH), jnp.int32),
            pltpu.VMEM((2, CH, D), jnp.float32),
            pltpu.VMEM((2, CH, D), jnp.float32),
            pltpu.VMEM((2, CH, D), jnp.float32),
            pltpu.VMEM((2, CH, D), jnp.float32),
            pltpu.VMEM((2, CH, D), jnp.float32),
            pltpu.SemaphoreType.DMA,
            pltpu.SemaphoreType.DMA,
        ],
    )
    uid = user_ids.astype(jnp.int32).reshape(NW, nch, CH)
    iid = item_ids.astype(jnp.int32).reshape(NW, nch, CH)
    return f(uid, iid, ue_gmf, ie_gmf, ue_mlp, ie_mlp)


BB = 2048


def _mlp_body(pr, um, im, w1, b1, w2, b2, w3, b3, wo, bo, out):
    dot = functools.partial(jnp.dot, preferred_element_type=jnp.float32)
    bf = jnp.bfloat16
    w1b = w1[...].astype(bf)
    h = dot(um[...].astype(bf), w1b[:D]) + dot(im[...].astype(bf), w1b[D:])
    h = jnp.maximum(h + b1[...], 0.0)
    h = jnp.maximum(dot(h.astype(bf), w2[...].astype(bf)) + b2[...], 0.0)
    h = jnp.maximum(dot(h.astype(bf), w3[...].astype(bf)) + b3[...], 0.0)
    wob = wo[...].astype(bf)
    out[...] = (dot(pr[...].astype(bf), wob[:D])
                + dot(h.astype(bf), wob[D:]) + bo[0, 0])


def _tc_mlp(pr, um, im, W1, b1, W2, b2, W3, b3, Wo, bo):
    row = lambda i: (i, 0)
    zero = lambda i: (0, 0)
    nb = pr.shape[0]
    rows_spec = pl.BlockSpec((BB, D), row)
    out = pl.pallas_call(
        _mlp_body,
        grid=(nb // BB,),
        in_specs=[
            rows_spec, rows_spec, rows_spec,
            pl.BlockSpec((256, 256), zero),
            pl.BlockSpec((1, 256), zero),
            pl.BlockSpec((256, 128), zero),
            pl.BlockSpec((1, 128), zero),
            pl.BlockSpec((128, 64), zero),
            pl.BlockSpec((1, 64), zero),
            pl.BlockSpec((192, 1), zero),
            pl.BlockSpec((1, 1), zero),
        ],
        out_specs=pl.BlockSpec((BB, 1), row),
        out_shape=jax.ShapeDtypeStruct((nb, 1), jnp.float32),
        compiler_params=pltpu.CompilerParams(
            dimension_semantics=("arbitrary",)),
    )(pr, um, im, W1, b1.reshape(1, 256), W2, b2.reshape(1, 128), W3,
      b3.reshape(1, 64), Wo, bo.reshape(1, 1))
    return out[:, 0]


NSPLIT = 2


def kernel(user_ids, item_ids, ue_gmf, ie_gmf, ue_mlp, ie_mlp,
           W1, b1, W2, b2, W3, b3, Wo, bo):
    h = B // NSPLIT
    outs = []
    for k in range(NSPLIT):
        pr, um, im = _sc_gather(user_ids[k * h:(k + 1) * h],
                                item_ids[k * h:(k + 1) * h],
                                ue_gmf, ie_gmf, ue_mlp, ie_mlp)
        outs.append(_tc_mlp(pr, um, im, W1, b1, W2, b2, W3, b3, Wo, bo))
    return jnp.concatenate(outs) if NSPLIT > 1 else outs[0]


# speedup vs baseline: 1.0682x; 1.0682x over previous
import functools

import jax
import jax.numpy as jnp
from jax import lax
from jax.experimental import pallas as pl
from jax.experimental.pallas import tpu as pltpu
from jax.experimental.pallas import tpu_sc as plsc

B = 16384
D = 128
NC = 2
NS = 16
NW = NC * NS
BPW = B // NW
CH = 64
NCH = BPW // CH
LANES = 16


def _prod_chunk(u_ref, i_ref, p_ref):
    def row(r, _):
        for k in range(D // LANES):
            sl = pl.ds(LANES * k, LANES)
            p_ref[r, sl] = u_ref[r, sl] * i_ref[r, sl]
        return 0

    lax.fori_loop(0, CH, row, 0, unroll=2)


def _make_gather_body(nch):
    bpw = nch * CH
    return functools.partial(_gather_body_merged, nch, bpw)


def _gather_body_merged(NCH, BPW, uid_ref, iid_ref, ug_t, ig_t, um_t, im_t,
                        pr_o, um_o, im_o,
                        uidx_v, iidx_v, bu, bi, bp, bm, bn, gsem, ssem):
    wid = lax.axis_index("s") * NC + lax.axis_index("c")
    base = wid * BPW
    pltpu.sync_copy(uid_ref.at[wid], uidx_v)
    pltpu.sync_copy(iid_ref.at[wid], iidx_v)

    njobs = 2 * NCH
    gath = [None] * njobs
    scat = [None] * njobs

    def issue_gather(j):
        c, s = j // 2, (j // 2) % 2
        if j % 2 == 0:
            return [pltpu.async_copy(ug_t.at[uidx_v.at[c]], bu.at[s], gsem),
                    pltpu.async_copy(ig_t.at[iidx_v.at[c]], bi.at[s], gsem)]
        return [pltpu.async_copy(um_t.at[uidx_v.at[c]], bm.at[s], gsem),
                pltpu.async_copy(im_t.at[iidx_v.at[c]], bn.at[s], gsem)]

    def issue_scatter(j):
        c, s = j // 2, (j // 2) % 2
        dst = pl.ds(base + c * CH, CH)
        if j % 2 == 0:
            return [pltpu.async_copy(bp.at[s], pr_o.at[dst], ssem)]
        return [pltpu.async_copy(bm.at[s], um_o.at[dst], ssem),
                pltpu.async_copy(bn.at[s], im_o.at[dst], ssem)]

    gath[0] = issue_gather(0)
    if njobs > 1:
        gath[1] = issue_gather(1)
    for j in range(njobs):
        nj = j + 2
        if nj < njobs:
            if nj - 4 >= 0 and scat[nj - 4] is not None:
                for h in scat[nj - 4]:
                    h.wait()
                scat[nj - 4] = None
            gath[nj] = issue_gather(nj)
        for h in gath[j]:
            h.wait()
        if j % 2 == 0:
            s = (j // 2) % 2
            if j - 4 >= 0 and scat[j - 4] is not None:
                for h in scat[j - 4]:
                    h.wait()
                scat[j - 4] = None
            _prod_chunk(bu.at[s], bi.at[s], bp.at[s])
        scat[j] = issue_scatter(j)
    for hs in scat:
        if hs is not None:
            for h in hs:
                h.wait()


def _sc_gather(user_ids, item_ids, ue_gmf, ie_gmf, ue_mlp, ie_mlp):
    nb = user_ids.shape[0]
    nch = nb // (NW * CH)
    mesh = plsc.VectorSubcoreMesh(core_axis_name="c", subcore_axis_name="s",
                                  num_cores=NC, num_subcores=NS)
    f = pl.kernel(
        _make_gather_body(nch),
        out_type=[jax.ShapeDtypeStruct((nb, D), jnp.float32)] * 3,
        mesh=mesh,
        scratch_types=[
            pltpu.VMEM((nch, CH), jnp.int32),
            pltpu.VMEM((nch, CH), jnp.int32),
            pltpu.VMEM((2, CH, D), jnp.float32),
            pltpu.VMEM((2, CH, D), jnp.float32),
            pltpu.VMEM((2, CH, D), jnp.float32),
            pltpu.VMEM((2, CH, D), jnp.float32),
            pltpu.VMEM((2, CH, D), jnp.float32),
            pltpu.SemaphoreType.DMA,
            pltpu.SemaphoreType.DMA,
        ],
    )
    uid = user_ids.astype(jnp.int32).reshape(NW, nch, CH)
    iid = item_ids.astype(jnp.int32).reshape(NW, nch, CH)
    return f(uid, iid, ue_gmf, ie_gmf, ue_mlp, ie_mlp)


BB = 2048


def _mlp_body(pr, um, im, w1, b1, w2, b2, w3, b3, wo, bo, out):
    dot = functools.partial(jnp.dot, preferred_element_type=jnp.float32)
    bf = jnp.bfloat16
    w1b = w1[...].astype(bf)
    h = dot(um[...].astype(bf), w1b[:D]) + dot(im[...].astype(bf), w1b[D:])
    h = jnp.maximum(h + b1[...], 0.0)
    h = jnp.maximum(dot(h.astype(bf), w2[...].astype(bf)) + b2[...], 0.0)
    h = jnp.maximum(dot(h.astype(bf), w3[...].astype(bf)) + b3[...], 0.0)
    wob = wo[...].astype(bf)
    pred = (dot(pr[...].astype(bf), wob[:D])
            + dot(h.astype(bf), wob[D:]) + bo[0, 0])
    out[...] = pred.reshape(BB // D, D)


def _tc_mlp(pr, um, im, W1, b1, W2, b2, W3, b3, Wo, bo):
    row = lambda i: (i, 0)
    zero = lambda i: (0, 0)
    nb = pr.shape[0]
    rows_spec = pl.BlockSpec((BB, D), row)
    out = pl.pallas_call(
        _mlp_body,
        grid=(nb // BB,),
        in_specs=[
            rows_spec, rows_spec, rows_spec,
            pl.BlockSpec((256, 256), zero),
            pl.BlockSpec((1, 256), zero),
            pl.BlockSpec((256, 128), zero),
            pl.BlockSpec((1, 128), zero),
            pl.BlockSpec((128, 64), zero),
            pl.BlockSpec((1, 64), zero),
            pl.BlockSpec((192, 1), zero),
            pl.BlockSpec((1, 1), zero),
        ],
        out_specs=pl.BlockSpec((BB // D, D), row),
        out_shape=jax.ShapeDtypeStruct((nb // D, D), jnp.float32),
        compiler_params=pltpu.CompilerParams(
            dimension_semantics=("arbitrary",)),
    )(pr, um, im, W1, b1.reshape(1, 256), W2, b2.reshape(1, 128), W3,
      b3.reshape(1, 64), Wo, bo.reshape(1, 1))
    return out.reshape(nb)


NSPLIT = 2


def kernel(user_ids, item_ids, ue_gmf, ie_gmf, ue_mlp, ie_mlp,
           W1, b1, W2, b2, W3, b3, Wo, bo):
    h = B // NSPLIT
    outs = []
    for k in range(NSPLIT):
        pr, um, im = _sc_gather(user_ids[k * h:(k + 1) * h],
                                item_ids[k * h:(k + 1) * h],
                                ue_gmf, ie_gmf, ue_mlp, ie_mlp)
        outs.append(_tc_mlp(pr, um, im, W1, b1, W2, b2, W3, b3, Wo, bo))
    return jnp.concatenate(outs) if NSPLIT > 1 else outs[0]


# speedup vs baseline: 1.0856x; 1.0163x over previous
import functools

import jax
import jax.numpy as jnp
from jax import lax
from jax.experimental import pallas as pl
from jax.experimental.pallas import tpu as pltpu
from jax.experimental.pallas import tpu_sc as plsc

B = 16384
D = 128
NC = 2
NS = 16
NW = NC * NS
BPW = B // NW
CH = 64
NCH = BPW // CH
LANES = 16


def _prod_chunk(u_ref, i_ref, p_ref):
    def row(r, _):
        for k in range(D // LANES):
            sl = pl.ds(LANES * k, LANES)
            p_ref[r, sl] = u_ref[r, sl] * i_ref[r, sl]
        return 0

    lax.fori_loop(0, CH, row, 0, unroll=2)


def _make_gather_body(nch, split):
    bpw = nch * CH
    return functools.partial(_gather_body_merged, nch, bpw, split)


def _gather_body_merged(NCH, BPW, SPLIT, uid_ref, iid_ref,
                        ug_t, ig_t, um_t, im_t,
                        pr_o, um_o, im_o,
                        uidx_v, iidx_v, bu, bi, bp, bm, bn, gsem, ssem):
    wid = lax.axis_index("s") * NC + lax.axis_index("c")
    base = wid * BPW
    pltpu.sync_copy(uid_ref.at[SPLIT, wid], uidx_v)
    pltpu.sync_copy(iid_ref.at[SPLIT, wid], iidx_v)

    njobs = 2 * NCH
    gath = [None] * njobs
    scat = [None] * njobs

    def issue_gather(j):
        c, s = j // 2, (j // 2) % 2
        if j % 2 == 0:
            return [pltpu.async_copy(ug_t.at[uidx_v.at[c]], bu.at[s], gsem),
                    pltpu.async_copy(ig_t.at[iidx_v.at[c]], bi.at[s], gsem)]
        return [pltpu.async_copy(um_t.at[uidx_v.at[c]], bm.at[s], gsem),
                pltpu.async_copy(im_t.at[iidx_v.at[c]], bn.at[s], gsem)]

    def issue_scatter(j):
        c, s = j // 2, (j // 2) % 2
        dst = pl.ds(base + c * CH, CH)
        if j % 2 == 0:
            return [pltpu.async_copy(bp.at[s], pr_o.at[dst], ssem)]
        return [pltpu.async_copy(bm.at[s], um_o.at[dst], ssem),
                pltpu.async_copy(bn.at[s], im_o.at[dst], ssem)]

    gath[0] = issue_gather(0)
    if njobs > 1:
        gath[1] = issue_gather(1)
    for j in range(njobs):
        nj = j + 2
        if nj < njobs:
            if nj - 4 >= 0 and scat[nj - 4] is not None:
                for h in scat[nj - 4]:
                    h.wait()
                scat[nj - 4] = None
            gath[nj] = issue_gather(nj)
        for h in gath[j]:
            h.wait()
        if j % 2 == 0:
            s = (j // 2) % 2
            if j - 4 >= 0 and scat[j - 4] is not None:
                for h in scat[j - 4]:
                    h.wait()
                scat[j - 4] = None
            _prod_chunk(bu.at[s], bi.at[s], bp.at[s])
        scat[j] = issue_scatter(j)
    for hs in scat:
        if hs is not None:
            for h in hs:
                h.wait()


def _sc_gather(uid4, iid4, split, ue_gmf, ie_gmf, ue_mlp, ie_mlp):
    nch = uid4.shape[2]
    nb = NW * nch * CH
    mesh = plsc.VectorSubcoreMesh(core_axis_name="c", subcore_axis_name="s",
                                  num_cores=NC, num_subcores=NS)
    f = pl.kernel(
        _make_gather_body(nch, split),
        out_type=[jax.ShapeDtypeStruct((nb, D), jnp.float32)] * 3,
        mesh=mesh,
        scratch_types=[
            pltpu.VMEM((nch, CH), jnp.int32),
            pltpu.VMEM((nch, CH), jnp.int32),
            pltpu.VMEM((2, CH, D), jnp.float32),
            pltpu.VMEM((2, CH, D), jnp.float32),
            pltpu.VMEM((2, CH, D), jnp.float32),
            pltpu.VMEM((2, CH, D), jnp.float32),
            pltpu.VMEM((2, CH, D), jnp.float32),
            pltpu.SemaphoreType.DMA,
            pltpu.SemaphoreType.DMA,
        ],
    )
    return f(uid4, iid4, ue_gmf, ie_gmf, ue_mlp, ie_mlp)


BB = 2048


def _mlp_body(pr, um, im, w1, b1, w2, b2, w3, b3, wo, bo, out):
    dot = functools.partial(jnp.dot, preferred_element_type=jnp.float32)
    bf = jnp.bfloat16
    w1b = w1[...].astype(bf)
    h = dot(um[...].astype(bf), w1b[:D]) + dot(im[...].astype(bf), w1b[D:])
    h = jnp.maximum(h + b1[...], 0.0)
    h = jnp.maximum(dot(h.astype(bf), w2[...].astype(bf)) + b2[...], 0.0)
    h = jnp.maximum(dot(h.astype(bf), w3[...].astype(bf)) + b3[...], 0.0)
    wob = wo[...].astype(bf)
    pred = (dot(pr[...].astype(bf), wob[:D])
            + dot(h.astype(bf), wob[D:]) + bo[0, 0])
    out[...] = pred.reshape(BB // D, D)


def _tc_mlp(pr, um, im, W1, b1, W2, b2, W3, b3, Wo, bo):
    row = lambda i: (i, 0)
    zero = lambda i: (0, 0)
    nb = pr.shape[0]
    rows_spec = pl.BlockSpec((BB, D), row)
    out = pl.pallas_call(
        _mlp_body,
        grid=(nb // BB,),
        in_specs=[
            rows_spec, rows_spec, rows_spec,
            pl.BlockSpec((256, 256), zero),
            pl.BlockSpec((1, 256), zero),
            pl.BlockSpec((256, 128), zero),
            pl.BlockSpec((1, 128), zero),
            pl.BlockSpec((128, 64), zero),
            pl.BlockSpec((1, 64), zero),
            pl.BlockSpec((192, 1), zero),
            pl.BlockSpec((1, 1), zero),
        ],
        out_specs=pl.BlockSpec((BB // D, D), row),
        out_shape=jax.ShapeDtypeStruct((nb // D, D), jnp.float32),
        compiler_params=pltpu.CompilerParams(
            dimension_semantics=("arbitrary",)),
    )(pr, um, im, W1, b1.reshape(1, 256), W2, b2.reshape(1, 128), W3,
      b3.reshape(1, 64), Wo, bo.reshape(1, 1))
    return out.reshape(nb)


NSPLIT = 2


def kernel(user_ids, item_ids, ue_gmf, ie_gmf, ue_mlp, ie_mlp,
           W1, b1, W2, b2, W3, b3, Wo, bo):
    nch = B // (NSPLIT * NW * CH)
    uid4 = user_ids.astype(jnp.int32).reshape(NSPLIT, NW, nch, CH)
    iid4 = item_ids.astype(jnp.int32).reshape(NSPLIT, NW, nch, CH)
    outs = []
    for k in range(NSPLIT):
        pr, um, im = _sc_gather(uid4, iid4, k, ue_gmf, ie_gmf,
                                ue_mlp, ie_mlp)
        outs.append(_tc_mlp(pr, um, im, W1, b1, W2, b2, W3, b3, Wo, bo))
    return jnp.concatenate(outs) if NSPLIT > 1 else outs[0]
